# baseline (speedup 1.0000x reference)
import jax
import jax.numpy as jnp
from jax.experimental import pallas as pl
from jax.experimental.pallas import tpu as pltpu

_T, _V, _F, _H = 50, 64, 8, 64
_TB = 10
_TP = _TB // 2
_E = _V * (_V - 1)


def _elu(x):
    return jnp.where(x > 0, x, jnp.exp(x) - 1.0)


def _bd(w):
    z = jnp.zeros_like(w)
    return jnp.concatenate([jnp.concatenate([w, z], axis=1),
                            jnp.concatenate([z, w], axis=1)], axis=0)


def _body(x_ref, m_ref, w1a_r, b1a_r, w1b_r, b1b_r, w2a_r, b2a_r, w2b_r, b2b_r,
          w3a_r, b3a_r, w3b_r, b3b_r, w4a_r, b4a_r, w4b_r, b4b_r, out_ref):
    f32 = jnp.float32
    dot = lambda a, b: jax.lax.dot(a, b, preferred_element_type=f32)
    h2w = 2 * _H
    dup = lambda b: jnp.concatenate([b, b], axis=1)

    w1a, w1b = _bd(w1a_r[...]), _bd(w1b_r[...])
    b1a, b1b = dup(b1a_r[...]), dup(b1b_r[...])
    w2a = w2a_r[...]
    w2as, w2ar = _bd(w2a[:_H]), _bd(w2a[_H:])
    w2b, b2a, b2b = _bd(w2b_r[...]), dup(b2a_r[...]), dup(b2b_r[...])
    w3a, w3b = _bd(w3a_r[...]), _bd(w3b_r[...])
    b3a, b3b = dup(b3a_r[...]), dup(b3b_r[...])
    w4a = w4a_r[...]
    w4as, w4ar, w4ak = _bd(w4a[:_H]), _bd(w4a[_H:2 * _H]), _bd(w4a[2 * _H:])
    w4b, b4a, b4b = _bd(w4b_r[...]), dup(b4a_r[...]), dup(b4b_r[...])

    xm = x_ref[0] * m_ref[...].reshape(_TB, _V, 1)
    xr = xm.reshape(_TP, 2, _V, _F)
    x = jnp.concatenate([xr[:, 0], xr[:, 1]], axis=-1).reshape(_TP * _V, 2 * _F)

    x1 = _elu(dot(x, w1a) + b1a)
    x1 = _elu(dot(x1, w1b) + b1b)

    a2 = (dot(x1, w2as) + b2a).reshape(_TP, _V, 1, h2w)
    b2 = dot(x1, w2ar).reshape(_TP, 1, _V, h2w)
    h2 = _elu(a2 + b2)
    x2 = _elu(dot(h2.reshape(_TP * _V * _V, h2w), w2b) + b2b)
    g2 = x2.reshape(_TP, _V, _V, h2w)

    s_ids = jax.lax.broadcasted_iota(jnp.int32, (1, _V, _V, h2w), 1)
    r_ids = jax.lax.broadcasted_iota(jnp.int32, (1, _V, _V, h2w), 2)
    masked = jnp.where(s_ids != r_ids, g2, 0.0)
    agg = jnp.sum(masked, axis=1).reshape(_TP * _V, h2w)

    x3 = _elu(dot(agg, w3a) + b3a)
    x3 = _elu(dot(x3, w3b) + b3b)

    c4 = (dot(x3, w4as) + b4a).reshape(_TP, _V, 1, h2w)
    d4 = dot(x3, w4ar).reshape(_TP, 1, _V, h2w)
    e4 = dot(x2, w4ak).reshape(_TP, _V, _V, h2w)
    h4 = _elu(c4 + d4 + e4)
    o = _elu(dot(h4.reshape(_TP * _V * _V, h2w), w4b) + b4b)
    o = o.reshape(_TP, _V, _V, h2w)

    jj = jax.lax.broadcasted_iota(jnp.int32, (1, _V, _V - 1, h2w), 2)
    ss = jax.lax.broadcasted_iota(jnp.int32, (1, _V, _V - 1, h2w), 1)
    outp = jnp.where(jj < ss, o[:, :, :_V - 1, :], o[:, :, 1:, :])

    ev = outp[:, :, :, :_H].reshape(_TP * _E, _H)
    od = outp[:, :, :, _H:].reshape(_TP * _E, _H)
    for p in range(_TP):
        out_ref[pl.ds(2 * p * _E, _E), :] = ev[p * _E:(p + 1) * _E]
        out_ref[pl.ds((2 * p + 1) * _E, _E), :] = od[p * _E:(p + 1) * _E]


def kernel(inputs, node_masks, all_node_inds, all_graph_info,
           w1a, b1a, w1b, b1b, w2a, b2a, w2b, b2b,
           w3a, b3a, w3b, b3b, w4a, b4a, w4b, b4b):
    b, t, v, f = inputs.shape
    h = w1b.shape[-1]

    row = lambda z: z.reshape(1, h)
    wspec = lambda s: pl.BlockSpec(s, lambda i: (0, 0))
    args = [
        inputs, node_masks.reshape(t * v, 1),
        w1a, row(b1a), w1b, row(b1b),
        w2a, row(b2a), w2b, row(b2b),
        w3a, row(b3a), w3b, row(b3b),
        w4a, row(b4a), w4b, row(b4b),
    ]
    in_specs = [pl.BlockSpec((1, _TB, v, f), lambda i: (0, i, 0, 0)),
                pl.BlockSpec((_TB * v, 1), lambda i: (i, 0))]
    in_specs += [wspec(a.shape) for a in args[2:]]

    return pl.pallas_call(
        _body,
        grid=(t // _TB,),
        in_specs=in_specs,
        out_specs=pl.BlockSpec((_TB * v * (v - 1), h), lambda i: (i, 0)),
        out_shape=jax.ShapeDtypeStruct((t * v * (v - 1), h), jnp.float32),
        compiler_params=pltpu.CompilerParams(
            dimension_semantics=("parallel",),
        ),
    )(*args)


# speedup vs baseline: 1.0111x; 1.0111x over previous
import jax
import jax.numpy as jnp
from jax.experimental import pallas as pl
from jax.experimental.pallas import tpu as pltpu

_T, _V, _F, _H = 50, 64, 8, 64
_TB = 10
_TP = _TB // 2
_E = _V * (_V - 1)


def _elu(x):
    return jnp.where(x > 0, x, jnp.exp(x) - 1.0)


def _bd(w):
    z = jnp.zeros_like(w)
    return jnp.concatenate([jnp.concatenate([w, z], axis=1),
                            jnp.concatenate([z, w], axis=1)], axis=0)


def _body(x_ref, w1a_r, b1a_r, w1b_r, b1b_r, w2a_r, b2a_r, w2b_r, b2b_r,
          w3a_r, b3a_r, w3b_r, b3b_r, w4a_r, b4a_r, w4b_r, b4b_r, out_ref):
    f32 = jnp.float32
    dot = lambda a, b: jax.lax.dot(a, b, preferred_element_type=f32)
    h2w = 2 * _H
    dup = lambda b: jnp.concatenate([b, b], axis=1)

    w1a, w1b = _bd(w1a_r[...]), _bd(w1b_r[...])
    b1a, b1b = dup(b1a_r[...]), dup(b1b_r[...])
    w2a = w2a_r[...]
    w2as, w2ar = _bd(w2a[:_H]), _bd(w2a[_H:])
    w2b, b2a, b2b = _bd(w2b_r[...]), dup(b2a_r[...]), dup(b2b_r[...])
    w3a, w3b = _bd(w3a_r[...]), _bd(w3b_r[...])
    b3a, b3b = dup(b3a_r[...]), dup(b3b_r[...])
    w4a = w4a_r[...]
    w4as, w4ar, w4ak = _bd(w4a[:_H]), _bd(w4a[_H:2 * _H]), _bd(w4a[2 * _H:])
    w4b, b4a, b4b = _bd(w4b_r[...]), dup(b4a_r[...]), dup(b4b_r[...])

    xr = x_ref[...].reshape(_TP, 2, _V, _F)
    x = jnp.concatenate([xr[:, 0], xr[:, 1]], axis=-1).reshape(_TP * _V, 2 * _F)

    x1 = _elu(dot(x, w1a) + b1a)
    x1 = _elu(dot(x1, w1b) + b1b)

    a2 = (dot(x1, w2as) + b2a).reshape(_TP, _V, 1, h2w)
    b2 = dot(x1, w2ar).reshape(_TP, 1, _V, h2w)
    h2 = _elu(a2 + b2)
    x2 = _elu(dot(h2.reshape(_TP * _V * _V, h2w), w2b) + b2b)
    g2 = x2.reshape(_TP, _V, _V, h2w)

    s_ids = jax.lax.broadcasted_iota(jnp.int32, (1, _V, _V, h2w), 1)
    r_ids = jax.lax.broadcasted_iota(jnp.int32, (1, _V, _V, h2w), 2)
    masked = jnp.where(s_ids != r_ids, g2, 0.0)
    agg = jnp.sum(masked, axis=1).reshape(_TP * _V, h2w)

    x3 = _elu(dot(agg, w3a) + b3a)
    x3 = _elu(dot(x3, w3b) + b3b)

    c4 = (dot(x3, w4as) + b4a).reshape(_TP, _V, 1, h2w)
    d4 = dot(x3, w4ar).reshape(_TP, 1, _V, h2w)
    e4 = dot(x2, w4ak).reshape(_TP, _V, _V, h2w)
    h4 = _elu(c4 + d4 + e4)
    o = _elu(dot(h4.reshape(_TP * _V * _V, h2w), w4b) + b4b)
    o = o.reshape(_TP, _V, _V, h2w)

    jj = jax.lax.broadcasted_iota(jnp.int32, (1, _V, _V - 1, h2w), 2)
    ss = jax.lax.broadcasted_iota(jnp.int32, (1, _V, _V - 1, h2w), 1)
    outp = jnp.where(jj < ss, o[:, :, :_V - 1, :], o[:, :, 1:, :])

    ev = outp[:, :, :, :_H].reshape(_TP * _E, _H)
    od = outp[:, :, :, _H:].reshape(_TP * _E, _H)
    for p in range(_TP):
        out_ref[pl.ds(2 * p * _E, _E), :] = ev[p * _E:(p + 1) * _E]
        out_ref[pl.ds((2 * p + 1) * _E, _E), :] = od[p * _E:(p + 1) * _E]


def kernel(inputs, node_masks, all_node_inds, all_graph_info,
           w1a, b1a, w1b, b1b, w2a, b2a, w2b, b2b,
           w3a, b3a, w3b, b3b, w4a, b4a, w4b, b4b):
    b, t, v, f = inputs.shape
    h = w1b.shape[-1]
    x = inputs.reshape(t * v, f) * node_masks.reshape(t * v, 1)

    row = lambda z: z.reshape(1, h)
    wspec = lambda s: pl.BlockSpec(s, lambda i: (0, 0))
    args = [
        x,
        w1a, row(b1a), w1b, row(b1b),
        w2a, row(b2a), w2b, row(b2b),
        w3a, row(b3a), w3b, row(b3b),
        w4a, row(b4a), w4b, row(b4b),
    ]
    in_specs = [pl.BlockSpec((_TB * v, f), lambda i: (i, 0))]
    in_specs += [wspec(a.shape) for a in args[1:]]

    return pl.pallas_call(
        _body,
        grid=(t // _TB,),
        in_specs=in_specs,
        out_specs=pl.BlockSpec((_TB * v * (v - 1), h), lambda i: (i, 0)),
        out_shape=jax.ShapeDtypeStruct((t * v * (v - 1), h), jnp.float32),
        compiler_params=pltpu.CompilerParams(
            dimension_semantics=("parallel",),
        ),
    )(*args)
